# Initial kernel scaffold; baseline (speedup 1.0000x reference)
import jax
import jax.numpy as jnp
from jax.experimental import pallas as pl


def kernel(x, edge_index, edge_attr, W_s, a_src, a_dst, b_s, Wgz, bgz, Wgr, bgr, Wgh, bgh, Wlz, blz, Wlr, blr, Wlh, blh, P, Wq, Wk, Wv, Wl, bl):
    raise NotImplementedError("write your pallas kernel here")


# speedup vs baseline: 1.0036x; 1.0036x over previous
import functools
import jax
import jax.numpy as jnp
from jax.experimental import pallas as pl

T, N, F, E, NH, C = 8, 10000, 256, 160000, 8, 16
HD = F // NH
BN = 400


def _proj_body(x_ref, ws_ref, bs_ref, asrc_ref, adst_ref, h_ref, als_ref, ald_ref):
    x = x_ref[...]
    h = jnp.dot(x, ws_ref[...], preferred_element_type=jnp.float32) + bs_ref[...]
    h_ref[...] = h
    f_iota = jax.lax.broadcasted_iota(jnp.int32, (F, NH), 0)
    h_iota = jax.lax.broadcasted_iota(jnp.int32, (F, NH), 1)
    M = (f_iota // HD == h_iota).astype(jnp.float32)
    als_ref[...] = jnp.dot(h * asrc_ref[...], M, preferred_element_type=jnp.float32)
    ald_ref[...] = jnp.dot(h * adst_ref[...], M, preferred_element_type=jnp.float32)


def _gcn3_body(x_ref, wz_ref, wr_ref, wh_ref, bz_ref, br_ref, bh_ref,
               xz_ref, xr_ref, xh_ref):
    x = x_ref[...]
    xz_ref[...] = jnp.dot(x, wz_ref[...], preferred_element_type=jnp.float32) + bz_ref[...]
    xr_ref[...] = jnp.dot(x, wr_ref[...], preferred_element_type=jnp.float32) + br_ref[...]
    xh_ref[...] = jnp.dot(x, wh_ref[...], preferred_element_type=jnp.float32) + bh_ref[...]


def _gru_body(az_ref, ar_ref, ah_ref, hh_ref,
              wz1_ref, wz2_ref, bz_ref, wr1_ref, wr2_ref, br_ref,
              wh1_ref, wh2_ref, bh_ref, out_ref):
    hh = hh_ref[...]
    z = jax.nn.sigmoid(
        jnp.dot(az_ref[...], wz1_ref[...], preferred_element_type=jnp.float32)
        + jnp.dot(hh, wz2_ref[...], preferred_element_type=jnp.float32) + bz_ref[...])
    r = jax.nn.sigmoid(
        jnp.dot(ar_ref[...], wr1_ref[...], preferred_element_type=jnp.float32)
        + jnp.dot(hh, wr2_ref[...], preferred_element_type=jnp.float32) + br_ref[...])
    htil = jnp.tanh(
        jnp.dot(ah_ref[...], wh1_ref[...], preferred_element_type=jnp.float32)
        + jnp.dot(hh * r, wh2_ref[...], preferred_element_type=jnp.float32) + bh_ref[...])
    out_ref[...] = z * hh + (1.0 - z) * htil


def _tattn_body(st_ref, p_ref, wq_ref, wk_ref, wv_ref, wl_ref, bl_ref, out_ref):
    st = st_ref[...]
    xt = st + p_ref[...][None]
    x2 = xt.reshape(BN * T, F)
    q = jnp.dot(x2, wq_ref[...], preferred_element_type=jnp.float32).reshape(BN, T, F)
    k = jnp.dot(x2, wk_ref[...], preferred_element_type=jnp.float32).reshape(BN, T, F)
    v = jnp.dot(x2, wv_ref[...], preferred_element_type=jnp.float32).reshape(BN, T, F)
    f_iota = jax.lax.broadcasted_iota(jnp.int32, (F, NH), 0)
    h_iota = jax.lax.broadcasted_iota(jnp.int32, (F, NH), 1)
    M = (f_iota // HD == h_iota).astype(jnp.float32)
    Mt = M.T
    scale = 1.0 / (HD ** 0.5)
    for t in range(T):
        qt = q[:, t, :]
        scores = []
        for s in range(t + 1):
            prod = qt * k[:, s, :]
            scores.append(jnp.dot(prod, M, preferred_element_type=jnp.float32) * scale)
        m = scores[0]
        for s in range(1, t + 1):
            m = jnp.maximum(m, scores[s])
        exps = [jnp.exp(sc - m) for sc in scores]
        denom = exps[0]
        for s in range(1, t + 1):
            denom = denom + exps[s]
        acc = jnp.zeros((BN, F), dtype=jnp.float32)
        for s in range(t + 1):
            a_exp = jnp.dot(exps[s] / denom, Mt, preferred_element_type=jnp.float32)
            acc = acc + a_exp * v[:, s, :]
        tout = jax.nn.relu(acc + st[:, t, :])
        out_ref[:, t, :] = jnp.dot(tout, wl_ref[...], preferred_element_type=jnp.float32) + bl_ref[...]


def _nf_spec():
    return pl.BlockSpec((BN, F), lambda i: (i, 0))


def _full_spec(shape):
    nd = len(shape)
    return pl.BlockSpec(shape, lambda i: (0,) * nd)


@jax.jit
def kernel(x, edge_index, edge_attr, W_s, a_src, a_dst, b_s, Wgz, bgz, Wgr, bgr,
           Wgh, bgh, Wlz, blz, Wlr, blr, Wlh, blh, P, Wq, Wk, Wv, Wl, bl):
    src = edge_index[0]
    dst = edge_index[1]
    deg_out = jax.ops.segment_sum(edge_attr, src, num_segments=N)
    deg_in = jax.ops.segment_sum(edge_attr, dst, num_segments=N)
    norm = edge_attr / (jnp.sqrt(deg_out[src] * deg_in[dst]) + 1e-6)

    grid = (N // BN,)

    proj = pl.pallas_call(
        _proj_body,
        grid=grid,
        in_specs=[_nf_spec(), _full_spec((F, F)), _full_spec((F,)),
                  _full_spec((1, F)), _full_spec((1, F))],
        out_specs=[_nf_spec(), pl.BlockSpec((BN, NH), lambda i: (i, 0)),
                   pl.BlockSpec((BN, NH), lambda i: (i, 0))],
        out_shape=[jax.ShapeDtypeStruct((N, F), jnp.float32),
                   jax.ShapeDtypeStruct((N, NH), jnp.float32),
                   jax.ShapeDtypeStruct((N, NH), jnp.float32)],
    )

    gcn3 = pl.pallas_call(
        _gcn3_body,
        grid=grid,
        in_specs=[_nf_spec()] + [_full_spec((F, F))] * 3 + [_full_spec((F,))] * 3,
        out_specs=[_nf_spec()] * 3,
        out_shape=[jax.ShapeDtypeStruct((N, F), jnp.float32)] * 3,
    )

    gru = pl.pallas_call(
        _gru_body,
        grid=grid,
        in_specs=[_nf_spec()] * 4
        + [_full_spec((F, F)), _full_spec((F, F)), _full_spec((F,))] * 3,
        out_specs=_nf_spec(),
        out_shape=jax.ShapeDtypeStruct((N, F), jnp.float32),
    )

    tattn = pl.pallas_call(
        _tattn_body,
        grid=grid,
        in_specs=[pl.BlockSpec((BN, T, F), lambda i: (i, 0, 0)),
                  _full_spec((T, F)), _full_spec((F, F)), _full_spec((F, F)),
                  _full_spec((F, F)), _full_spec((F, C)), _full_spec((C,))],
        out_specs=pl.BlockSpec((BN, T, C), lambda i: (i, 0, 0)),
        out_shape=jax.ShapeDtypeStruct((N, T, C), jnp.float32),
    )

    Wlz1, Wlz2 = Wlz[:F], Wlz[F:]
    Wlr1, Wlr2 = Wlr[:F], Wlr[F:]
    Wlh1, Wlh2 = Wlh[:F], Wlh[F:]

    Hh = jnp.zeros((N, F), dtype=x.dtype)
    st_list = []
    coe1 = []
    for t in range(T):
        xt = x[t]
        h, al_src, al_dst = proj(xt, W_s, b_s,
                                 a_src.reshape(1, F), a_dst.reshape(1, F))
        score = jax.nn.leaky_relu(al_src[src] + al_dst[dst], 0.2) * edge_attr[:, None]
        m = jax.ops.segment_max(score, dst, num_segments=N)
        m = jnp.where(jnp.isfinite(m), m, 0.0)
        e = jnp.exp(score - m[dst])
        ssum = jax.ops.segment_sum(e, dst, num_segments=N)
        attn = e / (ssum[dst] + 1e-16)
        hh = h.reshape(N, NH, HD)
        agg = jax.ops.segment_sum(attn[:, :, None] * hh[src], dst, num_segments=N)
        out1 = jax.nn.relu(agg.reshape(N, F) + h)
        coe1.append(attn)

        xz, xr, xh = gcn3(out1, Wgz, Wgr, Wgh, bgz, bgr, bgh)
        az = jax.ops.segment_sum(norm[:, None] * xz[src], dst, num_segments=N)
        ar = jax.ops.segment_sum(norm[:, None] * xr[src], dst, num_segments=N)
        ah = jax.ops.segment_sum(norm[:, None] * xh[src], dst, num_segments=N)
        Hh = gru(az, ar, ah, Hh, Wlz1, Wlz2, blz, Wlr1, Wlr2, blr, Wlh1, Wlh2, blh)
        st_list.append(Hh)

    st = jnp.stack(st_list, axis=1)
    logits_ntc = tattn(st, P, Wq, Wk, Wv, Wl, bl)
    logits = logits_ntc.transpose(1, 0, 2)
    return logits, jnp.stack(coe1, axis=0)


# speedup vs baseline: 1.0983x; 1.0944x over previous
import functools
import jax
import jax.numpy as jnp
from jax.experimental import pallas as pl

T, N, F, E, NH, C = 8, 10000, 256, 160000, 8, 16
HD = F // NH
BN = 400


def _proj_body(x_ref, ws_ref, bs_ref, asrc_ref, adst_ref, h_ref, als_ref, ald_ref):
    x = x_ref[...]
    h = jnp.dot(x, ws_ref[...], preferred_element_type=jnp.float32) + bs_ref[...]
    h_ref[...] = h
    f_iota = jax.lax.broadcasted_iota(jnp.int32, (F, NH), 0)
    h_iota = jax.lax.broadcasted_iota(jnp.int32, (F, NH), 1)
    M = (f_iota // HD == h_iota).astype(jnp.float32)
    als_ref[...] = jnp.dot(h * asrc_ref[...], M, preferred_element_type=jnp.float32)
    ald_ref[...] = jnp.dot(h * adst_ref[...], M, preferred_element_type=jnp.float32)


def _gru_body(s1_ref, rs_ref, hh_ref,
              wz_ref, cz_ref, wz2_ref, bz_ref,
              wr_ref, cr_ref, wr2_ref, br_ref,
              wh_ref, ch_ref, wh2_ref, bh_ref, out_ref):
    s1 = s1_ref[...]
    rs = rs_ref[...]
    hh = hh_ref[...]
    z = jax.nn.sigmoid(
        jnp.dot(s1, wz_ref[...], preferred_element_type=jnp.float32)
        + rs * cz_ref[...]
        + jnp.dot(hh, wz2_ref[...], preferred_element_type=jnp.float32) + bz_ref[...])
    r = jax.nn.sigmoid(
        jnp.dot(s1, wr_ref[...], preferred_element_type=jnp.float32)
        + rs * cr_ref[...]
        + jnp.dot(hh, wr2_ref[...], preferred_element_type=jnp.float32) + br_ref[...])
    htil = jnp.tanh(
        jnp.dot(s1, wh_ref[...], preferred_element_type=jnp.float32)
        + rs * ch_ref[...]
        + jnp.dot(hh * r, wh2_ref[...], preferred_element_type=jnp.float32) + bh_ref[...])
    out_ref[...] = z * hh + (1.0 - z) * htil


def _tattn_body(st_ref, p_ref, wq_ref, wk_ref, wv_ref, wl_ref, bl_ref, out_ref):
    st = st_ref[...]
    xt = st + p_ref[...][None]
    x2 = xt.reshape(BN * T, F)
    q = jnp.dot(x2, wq_ref[...], preferred_element_type=jnp.float32).reshape(BN, T, F)
    k = jnp.dot(x2, wk_ref[...], preferred_element_type=jnp.float32).reshape(BN, T, F)
    v = jnp.dot(x2, wv_ref[...], preferred_element_type=jnp.float32).reshape(BN, T, F)
    f_iota = jax.lax.broadcasted_iota(jnp.int32, (F, NH), 0)
    h_iota = jax.lax.broadcasted_iota(jnp.int32, (F, NH), 1)
    M = (f_iota // HD == h_iota).astype(jnp.float32)
    Mt = M.T
    scale = 1.0 / (HD ** 0.5)
    for t in range(T):
        qt = q[:, t, :]
        scores = []
        for s in range(t + 1):
            prod = qt * k[:, s, :]
            scores.append(jnp.dot(prod, M, preferred_element_type=jnp.float32) * scale)
        m = scores[0]
        for s in range(1, t + 1):
            m = jnp.maximum(m, scores[s])
        exps = [jnp.exp(sc - m) for sc in scores]
        denom = exps[0]
        for s in range(1, t + 1):
            denom = denom + exps[s]
        acc = jnp.zeros((BN, F), dtype=jnp.float32)
        for s in range(t + 1):
            a_exp = jnp.dot(exps[s] / denom, Mt, preferred_element_type=jnp.float32)
            acc = acc + a_exp * v[:, s, :]
        tout = jax.nn.relu(acc + st[:, t, :])
        out_ref[:, t, :] = jnp.dot(tout, wl_ref[...], preferred_element_type=jnp.float32) + bl_ref[...]


def _nf_spec():
    return pl.BlockSpec((BN, F), lambda i: (i, 0))


def _full_spec(shape):
    nd = len(shape)
    return pl.BlockSpec(shape, lambda i: (0,) * nd)


@jax.jit
def kernel(x, edge_index, edge_attr, W_s, a_src, a_dst, b_s, Wgz, bgz, Wgr, bgr,
           Wgh, bgh, Wlz, blz, Wlr, blr, Wlh, blh, P, Wq, Wk, Wv, Wl, bl):
    src = edge_index[0]
    dst = edge_index[1]
    deg_out = jax.ops.segment_sum(edge_attr, src, num_segments=N)
    deg_in = jax.ops.segment_sum(edge_attr, dst, num_segments=N)
    norm = edge_attr / (jnp.sqrt(deg_out[src] * deg_in[dst]) + 1e-6)

    grid = (N // BN,)

    proj = pl.pallas_call(
        _proj_body,
        grid=grid,
        in_specs=[_nf_spec(), _full_spec((F, F)), _full_spec((F,)),
                  _full_spec((1, F)), _full_spec((1, F))],
        out_specs=[_nf_spec(), pl.BlockSpec((BN, NH), lambda i: (i, 0)),
                   pl.BlockSpec((BN, NH), lambda i: (i, 0))],
        out_shape=[jax.ShapeDtypeStruct((N, F), jnp.float32),
                   jax.ShapeDtypeStruct((N, NH), jnp.float32),
                   jax.ShapeDtypeStruct((N, NH), jnp.float32)],
    )

    gru = pl.pallas_call(
        _gru_body,
        grid=grid,
        in_specs=[_nf_spec(), pl.BlockSpec((BN, 1), lambda i: (i, 0)), _nf_spec()]
        + [_full_spec((F, F)), _full_spec((1, F)), _full_spec((F, F)),
           _full_spec((F,))] * 3,
        out_specs=_nf_spec(),
        out_shape=jax.ShapeDtypeStruct((N, F), jnp.float32),
    )

    tattn = pl.pallas_call(
        _tattn_body,
        grid=grid,
        in_specs=[pl.BlockSpec((BN, T, F), lambda i: (i, 0, 0)),
                  _full_spec((T, F)), _full_spec((F, F)), _full_spec((F, F)),
                  _full_spec((F, F)), _full_spec((F, C)), _full_spec((C,))],
        out_specs=pl.BlockSpec((BN, T, C), lambda i: (i, 0, 0)),
        out_shape=jax.ShapeDtypeStruct((N, T, C), jnp.float32),
    )

    Wlz1, Wlz2 = Wlz[:F], Wlz[F:]
    Wlr1, Wlr2 = Wlr[:F], Wlr[F:]
    Wlh1, Wlh2 = Wlh[:F], Wlh[F:]
    Wz, cz = Wgz @ Wlz1, (bgz @ Wlz1).reshape(1, F)
    Wr, cr = Wgr @ Wlr1, (bgr @ Wlr1).reshape(1, F)
    Wh, ch = Wgh @ Wlh1, (bgh @ Wlh1).reshape(1, F)
    rowsum = jax.ops.segment_sum(norm, dst, num_segments=N).reshape(N, 1)

    Hh = jnp.zeros((N, F), dtype=x.dtype)
    st_list = []
    coe1 = []
    for t in range(T):
        xt = x[t]
        h, al_src, al_dst = proj(xt, W_s, b_s,
                                 a_src.reshape(1, F), a_dst.reshape(1, F))
        score = jax.nn.leaky_relu(al_src[src] + al_dst[dst], 0.2) * edge_attr[:, None]
        m = jax.ops.segment_max(score, dst, num_segments=N)
        m = jnp.where(jnp.isfinite(m), m, 0.0)
        e = jnp.exp(score - m[dst])
        ssum = jax.ops.segment_sum(e, dst, num_segments=N)
        attn = e / (ssum[dst] + 1e-16)
        hh = h.reshape(N, NH, HD)
        agg = jax.ops.segment_sum(attn[:, :, None] * hh[src], dst, num_segments=N)
        out1 = jax.nn.relu(agg.reshape(N, F) + h)
        coe1.append(attn)

        s1 = jax.ops.segment_sum(norm[:, None] * out1[src], dst, num_segments=N)
        Hh = gru(s1, rowsum, Hh, Wz, cz, Wlz2, blz, Wr, cr, Wlr2, blr,
                 Wh, ch, Wlh2, blh)
        st_list.append(Hh)

    st = jnp.stack(st_list, axis=1)
    logits_ntc = tattn(st, P, Wq, Wk, Wv, Wl, bl)
    logits = logits_ntc.transpose(1, 0, 2)
    return logits, jnp.stack(coe1, axis=0)
